# Initial kernel scaffold; baseline (speedup 1.0000x reference)
import functools

import jax
import jax.numpy as jnp
from jax import lax
from jax.experimental import pallas as pl
from jax.experimental.pallas import tpu as pltpu
from jax.experimental.pallas import tpu_sc as plsc

N = 10000
E = 320000
D = 128

NC = 2
NS = 16
LN = 16

K = 80
CH = E // (NC * NS * K)
RPW = N // NS

_MESH = plsc.VectorSubcoreMesh(
    core_axis_name="c", subcore_axis_name="s", num_cores=NC, num_subcores=NS
)


def _sc_body(has_t, write_e, *refs):
    it = iter(refs)
    src2 = next(it)
    dst2 = next(it)
    p1 = next(it)
    p2 = next(it)
    t = next(it) if has_t else None
    e_out = next(it) if write_e else None
    agg_a = next(it)
    agg_b = next(it)
    idxs = next(it)
    idxd = next(it)
    abuf = next(it)
    bbuf = next(it)
    tbuf = next(it)
    zbuf = next(it)
    agg_sh = next(it)
    sem_a = next(it)
    sem_b = next(it)
    sem_t = next(it)

    c = lax.axis_index("c")
    s = lax.axis_index("s")
    base_row = (c * NS + s) * CH

    pltpu.sync_copy(src2.at[pl.ds(base_row, CH)], idxs)
    pltpu.sync_copy(dst2.at[pl.ds(base_row, CH)], idxd)

    def _zrow(r, _):
        for j in range(D // LN):
            zbuf[r, pl.ds(j * LN, LN)] = jnp.zeros((LN,), jnp.float32)
        return 0

    lax.fori_loop(0, 125, _zrow, 0)
    for q in range(RPW // 125):
        pltpu.sync_copy(zbuf, agg_sh.at[pl.ds(s * RPW + q * 125, 125)])
    plsc.subcore_barrier()

    def _chunk(j, _):
        erow = (base_row + j) * K
        cp_a = pltpu.async_copy(p1.at[idxs.at[j]], abuf, sem_a)
        cp_b = pltpu.async_copy(p2.at[idxd.at[j]], bbuf, sem_b)
        if has_t:
            cp_t = pltpu.async_copy(t.at[pl.ds(erow, K)], tbuf, sem_t)
        cp_a.wait()
        cp_b.wait()
        if has_t:
            cp_t.wait()

        def _row(r, _):
            for jj in range(D // LN):
                sl = pl.ds(jj * LN, LN)
                v = abuf[r, sl] + bbuf[r, sl]
                if has_t:
                    v = v + tbuf[r, sl]
                abuf[r, sl] = jnp.maximum(v, 0.0)
            return 0

        lax.fori_loop(0, K, _row, 0)

        pltpu.sync_copy(abuf, agg_sh.at[idxd.at[j]], add=True)
        if write_e:
            pltpu.sync_copy(abuf, e_out.at[pl.ds(erow, K)])
        return 0

    lax.fori_loop(0, CH, _chunk, 0)
    plsc.subcore_barrier()

    @pl.when(c == 0)
    def _():
        pltpu.sync_copy(agg_sh.at[pl.ds(s * RPW, RPW)],
                        agg_a.at[pl.ds(s * RPW, RPW)])

    @pl.when(c == 1)
    def _():
        pltpu.sync_copy(agg_sh.at[pl.ds(s * RPW, RPW)],
                        agg_b.at[pl.ds(s * RPW, RPW)])


def _make_sc_kernel(has_t, write_e):
    f32 = jnp.float32
    outs = []
    if write_e:
        outs.append(jax.ShapeDtypeStruct((E, D), f32))
    outs.append(jax.ShapeDtypeStruct((N, D), f32))
    outs.append(jax.ShapeDtypeStruct((N, D), f32))
    scratch = [
        pltpu.VMEM((CH, K), jnp.int32),
        pltpu.VMEM((CH, K), jnp.int32),
        pltpu.VMEM((K, D), f32),
        pltpu.VMEM((K, D), f32),
        pltpu.VMEM((K, D), f32),
        pltpu.VMEM((125, D), f32),
        pltpu.VMEM_SHARED((N, D), f32),
        pltpu.SemaphoreType.DMA,
        pltpu.SemaphoreType.DMA,
        pltpu.SemaphoreType.DMA,
    ]
    return pl.kernel(
        functools.partial(_sc_body, has_t, write_e),
        out_type=tuple(outs),
        mesh=_MESH,
        scratch_types=scratch,
    )


_BMN = 2000
_BME = 4000


def _wspec():
    return pl.BlockSpec((D, D), lambda i: (0, 0))


def _bspec():
    return pl.BlockSpec((1, D), lambda i: (0, 0))


def _rows(bm):
    return pl.BlockSpec((bm, D), lambda i: (i, 0))


def _prep0_body(x_ref, w1_ref, w2_ref, be_ref, p1_ref, p2_ref):
    x = x_ref[...]
    p1_ref[...] = jnp.dot(x, w1_ref[...], preferred_element_type=jnp.float32)
    p2_ref[...] = (
        jnp.dot(x, w2_ref[...], preferred_element_type=jnp.float32)
        + be_ref[...]
    )


def _prep0(z, w1, w2, be):
    return pl.pallas_call(
        _prep0_body,
        grid=(N // _BMN,),
        in_specs=[_rows(_BMN), _wspec(), _wspec(), _bspec()],
        out_specs=[_rows(_BMN), _rows(_BMN)],
        out_shape=[jax.ShapeDtypeStruct((N, D), jnp.float32)] * 2,
    )(z, w1, w2, be)


def _edge_mm_body(e_ref, w_ref, o_ref):
    o_ref[...] = jnp.dot(
        e_ref[...], w_ref[...], preferred_element_type=jnp.float32
    )


def _edge_mm(e0, w3):
    return pl.pallas_call(
        _edge_mm_body,
        grid=(E // _BME,),
        in_specs=[_rows(_BME), _wspec()],
        out_specs=_rows(_BME),
        out_shape=jax.ShapeDtypeStruct((E, D), jnp.float32),
    )(e0, w3)


def _edge_mm2_body(e0_ref, e1_ref, w_ref, o_ref):
    o_ref[...] = jnp.dot(
        e0_ref[...] + e1_ref[...], w_ref[...],
        preferred_element_type=jnp.float32,
    )


def _edge_mm2(e0, e1, w3):
    return pl.pallas_call(
        _edge_mm2_body,
        grid=(E // _BME,),
        in_specs=[_rows(_BME), _rows(_BME), _wspec()],
        out_specs=_rows(_BME),
        out_shape=jax.ShapeDtypeStruct((E, D), jnp.float32),
    )(e0, e1, w3)


def _node_body(residual, prep, *refs):
    it = iter(refs)
    x_ref = next(it)
    aa_ref = next(it)
    ab_ref = next(it)
    wna_ref = next(it)
    wnb_ref = next(it)
    bn_ref = next(it)
    if prep:
        w1_ref = next(it)
        w2_ref = next(it)
        be_ref = next(it)
    h_ref = next(it)
    if prep:
        p1_ref = next(it)
        p2_ref = next(it)
    x = x_ref[...]
    agg = aa_ref[...] + ab_ref[...]
    h = jnp.maximum(
        jnp.dot(x, wna_ref[...], preferred_element_type=jnp.float32)
        + jnp.dot(agg, wnb_ref[...], preferred_element_type=jnp.float32)
        + bn_ref[...],
        0.0,
    )
    if residual:
        h = h + x
    h_ref[...] = h
    if prep:
        p1_ref[...] = jnp.dot(
            h, w1_ref[...], preferred_element_type=jnp.float32
        )
        p2_ref[...] = (
            jnp.dot(h, w2_ref[...], preferred_element_type=jnp.float32)
            + be_ref[...]
        )


def _node(residual, prep, x, agg_a, agg_b, wna, wnb, bn, *prep_args):
    n_out = 3 if prep else 1
    in_specs = [_rows(_BMN)] * 3 + [_wspec(), _wspec(), _bspec()]
    if prep:
        in_specs += [_wspec(), _wspec(), _bspec()]
    res = pl.pallas_call(
        functools.partial(_node_body, residual, prep),
        grid=(N // _BMN,),
        in_specs=in_specs,
        out_specs=[_rows(_BMN)] * n_out,
        out_shape=[jax.ShapeDtypeStruct((N, D), jnp.float32)] * n_out,
    )(x, agg_a, agg_b, wna, wnb, bn, *prep_args)
    return res if prep else res[0]


def kernel(edge_index, z, We0, be0, Wn0, bn0, We1, be1, Wn1, bn1,
           We2, be2, Wn2, bn2):
    src2 = edge_index[0].reshape(E // K, K)
    dst2 = edge_index[1].reshape(E // K, K)

    be0r = be0.reshape(1, D)
    be1r = be1.reshape(1, D)
    be2r = be2.reshape(1, D)
    bn0r = bn0.reshape(1, D)
    bn1r = bn1.reshape(1, D)
    bn2r = bn2.reshape(1, D)

    sc0 = _make_sc_kernel(has_t=False, write_e=True)
    sc1 = _make_sc_kernel(has_t=True, write_e=True)
    sc2 = _make_sc_kernel(has_t=True, write_e=False)

    p1, p2 = _prep0(z, We0[:D], We0[D:], be0r)
    e0, agg_a, agg_b = sc0(src2, dst2, p1, p2)
    x1, p1, p2 = _node(False, True, z, agg_a, agg_b,
                       Wn0[:D], Wn0[D:], bn0r, We1[:D], We1[D:2 * D], be1r)

    t1 = _edge_mm(e0, We1[2 * D:])
    e1, agg_a, agg_b = sc1(src2, dst2, p1, p2, t1)
    x2, p1, p2 = _node(True, True, x1, agg_a, agg_b,
                       Wn1[:D], Wn1[D:], bn1r, We2[:D], We2[D:2 * D], be2r)

    t2 = _edge_mm2(e0, e1, We2[2 * D:])
    agg_a, agg_b = sc2(src2, dst2, p1, p2, t2)
    out = _node(False, False, x2, agg_a, agg_b, Wn2[:D], Wn2[D:], bn2r)
    return out


# speedup vs baseline: 6.0358x; 6.0358x over previous
import functools

import jax
import jax.numpy as jnp
from jax import lax
from jax.experimental import pallas as pl
from jax.experimental.pallas import tpu as pltpu
from jax.experimental.pallas import tpu_sc as plsc

N = 10000
E = 320000
D = 128

NC = 2
NS = 16
LN = 16

K = 40
CH = E // (NC * NS * K)
IDXB = 64
RPW = 632
RPW_LAST = N - 15 * RPW

_MESH = plsc.VectorSubcoreMesh(
    core_axis_name="c", subcore_axis_name="s", num_cores=NC, num_subcores=NS
)


def _sc_body(has_t, write_e, *refs):
    it = iter(refs)
    src3 = next(it)
    dst3 = next(it)
    p1 = next(it)
    p2 = next(it)
    t = next(it) if has_t else None
    e_out = next(it) if write_e else None
    agg_a = next(it)
    agg_b = next(it)
    idxs = next(it)
    idxd = next(it)
    abufs = (next(it), next(it))
    bbufs = (next(it), next(it))
    tbufs = (next(it), next(it))
    agg_sh = next(it)
    sems_a = (next(it), next(it))
    sems_b = (next(it), next(it))
    sems_t = (next(it), next(it))

    c = lax.axis_index("c")
    s = lax.axis_index("s")
    wid = c * NS + s

    z0 = abufs[0]

    def _zrow(r, _):
        for j in range(D // LN):
            z0[r, pl.ds(j * LN, LN)] = jnp.zeros((LN,), jnp.float32)
        return 0

    lax.fori_loop(0, K, _zrow, 0)

    @pl.when(s < NS - 1)
    def _():
        for q in range(RPW // K):
            pltpu.sync_copy(z0, agg_sh.at[pl.ds(s * RPW + q * K, K)])
        rem = RPW % K
        pltpu.sync_copy(z0.at[pl.ds(0, rem)],
                        agg_sh.at[pl.ds(s * RPW + RPW - rem, rem)])

    @pl.when(s == NS - 1)
    def _():
        base = (NS - 1) * RPW
        for q in range(RPW_LAST // K):
            pltpu.sync_copy(z0, agg_sh.at[pl.ds(base + q * K, K)])

    plsc.subcore_barrier()

    def _issue(cl, base_c, slot):
        erow = (wid * CH + base_c + cl) * K
        pltpu.async_copy(p1.at[idxs.at[cl]], abufs[slot], sems_a[slot])
        pltpu.async_copy(p2.at[idxd.at[cl]], bbufs[slot], sems_b[slot])
        if has_t:
            pltpu.async_copy(t.at[pl.ds(erow, K)], tbufs[slot], sems_t[slot])

    def _wait(slot):
        dummy = p1.at[pl.ds(0, K)]
        pltpu.make_async_copy(dummy, abufs[slot], sems_a[slot]).wait()
        pltpu.make_async_copy(dummy, bbufs[slot], sems_b[slot]).wait()
        if has_t:
            pltpu.make_async_copy(dummy, tbufs[slot], sems_t[slot]).wait()

    def _compute(slot):
        ab, bb, tb = abufs[slot], bbufs[slot], tbufs[slot]

        def _row(r, _):
            for jj in range(D // LN):
                sl = pl.ds(jj * LN, LN)
                v = ab[r, sl] + bb[r, sl]
                if has_t:
                    v = v + tb[r, sl]
                ab[r, sl] = jnp.maximum(v, 0.0)
            return 0

        lax.fori_loop(0, K, _row, 0)

    blocks = []
    base_c = 0
    while base_c < CH:
        blocks.append((base_c, min(IDXB, CH - base_c)))
        base_c += IDXB

    for base_c, bsz in blocks:
        pltpu.sync_copy(src3.at[wid].at[pl.ds(base_c, bsz)],
                        idxs.at[pl.ds(0, bsz)])
        pltpu.sync_copy(dst3.at[wid].at[pl.ds(base_c, bsz)],
                        idxd.at[pl.ds(0, bsz)])
        _issue(0, base_c, 0)
        _issue(1, base_c, 1)

        def _pair(j2, _, base_c=base_c, bsz=bsz):
            for slot in (0, 1):
                cl = j2 * 2 + slot
                gl = base_c + cl
                erow = (wid * CH + gl) * K
                _wait(slot)
                _compute(slot)
                pltpu.sync_copy(abufs[slot], agg_sh.at[idxd.at[cl]],
                                add=True)
                if write_e:
                    pltpu.sync_copy(abufs[slot], e_out.at[pl.ds(erow, K)])

                @pl.when(cl + 2 < bsz)
                def _(cl=cl, slot=slot, base_c=base_c):
                    _issue(cl + 2, base_c, slot)
            return 0

        lax.fori_loop(0, bsz // 2, _pair, 0)

    plsc.subcore_barrier()

    for core, agg_out in ((0, agg_a), (1, agg_b)):
        @pl.when(jnp.logical_and(c == core, s < NS - 1))
        def _(agg_out=agg_out):
            pltpu.sync_copy(agg_sh.at[pl.ds(s * RPW, RPW)],
                            agg_out.at[pl.ds(s * RPW, RPW)])

        @pl.when(jnp.logical_and(c == core, s == NS - 1))
        def _(agg_out=agg_out):
            pltpu.sync_copy(agg_sh.at[pl.ds((NS - 1) * RPW, RPW_LAST)],
                            agg_out.at[pl.ds((NS - 1) * RPW, RPW_LAST)])


def _make_sc_kernel(has_t, write_e):
    f32 = jnp.float32
    outs = []
    if write_e:
        outs.append(jax.ShapeDtypeStruct((E, D), f32))
    outs.append(jax.ShapeDtypeStruct((N, D), f32))
    outs.append(jax.ShapeDtypeStruct((N, D), f32))
    scratch = (
        [pltpu.VMEM((IDXB, K), jnp.int32)] * 2
        + [pltpu.VMEM((K, D), f32)] * 6
        + [pltpu.VMEM_SHARED((N, D), f32)]
        + [pltpu.SemaphoreType.DMA] * 6
    )
    return pl.kernel(
        functools.partial(_sc_body, has_t, write_e),
        out_type=tuple(outs),
        mesh=_MESH,
        scratch_types=scratch,
    )


_BMN = 2000
_BME = 4000


def _wspec():
    return pl.BlockSpec((D, D), lambda i: (0, 0))


def _bspec():
    return pl.BlockSpec((1, D), lambda i: (0, 0))


def _rows(bm):
    return pl.BlockSpec((bm, D), lambda i: (i, 0))


def _prep0_body(x_ref, w1_ref, w2_ref, be_ref, p1_ref, p2_ref):
    x = x_ref[...]
    p1_ref[...] = jnp.dot(x, w1_ref[...], preferred_element_type=jnp.float32)
    p2_ref[...] = (
        jnp.dot(x, w2_ref[...], preferred_element_type=jnp.float32)
        + be_ref[...]
    )


def _prep0(z, w1, w2, be):
    return pl.pallas_call(
        _prep0_body,
        grid=(N // _BMN,),
        in_specs=[_rows(_BMN), _wspec(), _wspec(), _bspec()],
        out_specs=[_rows(_BMN), _rows(_BMN)],
        out_shape=[jax.ShapeDtypeStruct((N, D), jnp.float32)] * 2,
    )(z, w1, w2, be)


def _edge_mm_body(e_ref, w_ref, o_ref):
    o_ref[...] = jnp.dot(
        e_ref[...], w_ref[...], preferred_element_type=jnp.float32
    )


def _edge_mm(e0, w3):
    return pl.pallas_call(
        _edge_mm_body,
        grid=(E // _BME,),
        in_specs=[_rows(_BME), _wspec()],
        out_specs=_rows(_BME),
        out_shape=jax.ShapeDtypeStruct((E, D), jnp.float32),
    )(e0, w3)


def _edge_mm2_body(e0_ref, e1_ref, w_ref, o_ref):
    o_ref[...] = jnp.dot(
        e0_ref[...] + e1_ref[...], w_ref[...],
        preferred_element_type=jnp.float32,
    )


def _edge_mm2(e0, e1, w3):
    return pl.pallas_call(
        _edge_mm2_body,
        grid=(E // _BME,),
        in_specs=[_rows(_BME), _rows(_BME), _wspec()],
        out_specs=_rows(_BME),
        out_shape=jax.ShapeDtypeStruct((E, D), jnp.float32),
    )(e0, e1, w3)


def _node_body(residual, prep, *refs):
    it = iter(refs)
    x_ref = next(it)
    aa_ref = next(it)
    ab_ref = next(it)
    wna_ref = next(it)
    wnb_ref = next(it)
    bn_ref = next(it)
    if prep:
        w1_ref = next(it)
        w2_ref = next(it)
        be_ref = next(it)
    h_ref = next(it)
    if prep:
        p1_ref = next(it)
        p2_ref = next(it)
    x = x_ref[...]
    agg = aa_ref[...] + ab_ref[...]
    h = jnp.maximum(
        jnp.dot(x, wna_ref[...], preferred_element_type=jnp.float32)
        + jnp.dot(agg, wnb_ref[...], preferred_element_type=jnp.float32)
        + bn_ref[...],
        0.0,
    )
    if residual:
        h = h + x
    h_ref[...] = h
    if prep:
        p1_ref[...] = jnp.dot(
            h, w1_ref[...], preferred_element_type=jnp.float32
        )
        p2_ref[...] = (
            jnp.dot(h, w2_ref[...], preferred_element_type=jnp.float32)
            + be_ref[...]
        )


def _node(residual, prep, x, agg_a, agg_b, wna, wnb, bn, *prep_args):
    n_out = 3 if prep else 1
    in_specs = [_rows(_BMN)] * 3 + [_wspec(), _wspec(), _bspec()]
    if prep:
        in_specs += [_wspec(), _wspec(), _bspec()]
    res = pl.pallas_call(
        functools.partial(_node_body, residual, prep),
        grid=(N // _BMN,),
        in_specs=in_specs,
        out_specs=[_rows(_BMN)] * n_out,
        out_shape=[jax.ShapeDtypeStruct((N, D), jnp.float32)] * n_out,
    )(x, agg_a, agg_b, wna, wnb, bn, *prep_args)
    return res if prep else res[0]


def kernel(edge_index, z, We0, be0, Wn0, bn0, We1, be1, Wn1, bn1,
           We2, be2, Wn2, bn2):
    nw = NC * NS
    src3 = edge_index[0].reshape(nw, CH, K)
    dst3 = edge_index[1].reshape(nw, CH, K)

    be0r = be0.reshape(1, D)
    be1r = be1.reshape(1, D)
    be2r = be2.reshape(1, D)
    bn0r = bn0.reshape(1, D)
    bn1r = bn1.reshape(1, D)
    bn2r = bn2.reshape(1, D)

    sc0 = _make_sc_kernel(has_t=False, write_e=True)
    sc1 = _make_sc_kernel(has_t=True, write_e=True)
    sc2 = _make_sc_kernel(has_t=True, write_e=False)

    p1, p2 = _prep0(z, We0[:D], We0[D:], be0r)
    e0, agg_a, agg_b = sc0(src3, dst3, p1, p2)
    x1, p1, p2 = _node(False, True, z, agg_a, agg_b,
                       Wn0[:D], Wn0[D:], bn0r, We1[:D], We1[D:2 * D], be1r)

    t1 = _edge_mm(e0, We1[2 * D:])
    e1, agg_a, agg_b = sc1(src3, dst3, p1, p2, t1)
    x2, p1, p2 = _node(True, True, x1, agg_a, agg_b,
                       Wn1[:D], Wn1[D:], bn1r, We2[:D], We2[D:2 * D], be2r)

    t2 = _edge_mm2(e0, e1, We2[2 * D:])
    agg_a, agg_b = sc2(src3, dst3, p1, p2, t2)
    out = _node(False, False, x2, agg_a, agg_b, Wn2[:D], Wn2[D:], bn2r)
    return out
